# baseline (speedup 1.0000x reference)
import jax
import jax.numpy as jnp
from jax.experimental import pallas as pl
from jax.experimental.pallas import tpu as pltpu

_N = 6


def _partsum_kernel(x_ref, o_ref):
    BB, L, C = x_ref.shape
    ratio = L // _N
    for bb in range(BB):
        xb = x_ref[bb]
        parts = [
            jnp.sum(xb[i * ratio:(i + 1) * ratio, :], axis=0, keepdims=True)
            for i in range(_N)
        ]
        o_ref[bb] = jnp.concatenate(parts, axis=0)


def _finish_kernel(ps_ref, w_ref, b_ref, o_ref, *, ratio):
    B = ps_ref.shape[0]
    C = ps_ref.shape[2]
    pf = ps_ref[...] * (1.0 / ratio)

    diff = pf[:, :, None, :] - pf[:, None, :, :]
    d2 = jnp.sum(diff * diff, axis=-1)

    a = d2[:, :, :, None]
    bj = d2[:, :, None, :]
    jidx = jax.lax.broadcasted_iota(jnp.int32, (1, 1, _N, _N), 3)
    midx = jax.lax.broadcasted_iota(jnp.int32, (1, 1, _N, _N), 2)
    beats = (bj < a) | ((bj == a) & (jidx < midx))
    rank = jnp.sum(beats.astype(jnp.float32), axis=-1)
    adj = (rank <= 2.5).astype(jnp.float32)

    c = jnp.sum(adj, axis=1) * (1.0 / ((3.0 + 1e-6) * _N))
    g = jnp.sum(c[:, :, None] * pf, axis=1)
    mean_pf = jnp.sum(pf, axis=1) * (1.0 / _N)
    out = jax.lax.dot_general(
        g, w_ref[...], (((1,), (1,)), ((), ())),
        preferred_element_type=jnp.float32)
    o_ref[...] = out + b_ref[...] + mean_pf


@jax.jit
def kernel(x, W, b):
    B, L, C = x.shape
    BB = 8
    ps = pl.pallas_call(
        _partsum_kernel,
        grid=(B // BB,),
        in_specs=[pl.BlockSpec((BB, L, C), lambda i: (i, 0, 0))],
        out_specs=pl.BlockSpec((BB, _N, C), lambda i: (i, 0, 0)),
        out_shape=jax.ShapeDtypeStruct((B, _N, C), x.dtype),
        compiler_params=pltpu.CompilerParams(
            dimension_semantics=("arbitrary",),
        ),
    )(x)

    import functools
    out = pl.pallas_call(
        functools.partial(_finish_kernel, ratio=L // _N),
        in_specs=[
            pl.BlockSpec((B, _N, C), lambda: (0, 0, 0)),
            pl.BlockSpec((C, C), lambda: (0, 0)),
            pl.BlockSpec((1, C), lambda: (0, 0)),
        ],
        out_specs=pl.BlockSpec((B, C), lambda: (0, 0)),
        out_shape=jax.ShapeDtypeStruct((B, C), x.dtype),
    )(ps, W, b.reshape(1, C))
    return out


# speedup vs baseline: 1.0111x; 1.0111x over previous
import jax
import jax.numpy as jnp
from jax.experimental import pallas as pl
from jax.experimental.pallas import tpu as pltpu

_N = 6


def _partsum_kernel(x_ref, o_ref):
    BB, L, C = x_ref.shape
    ratio = L // _N
    for bb in range(BB):
        xb = x_ref[bb]
        parts = [
            jnp.sum(xb[i * ratio:(i + 1) * ratio, :], axis=0, keepdims=True)
            for i in range(_N)
        ]
        o_ref[bb] = jnp.concatenate(parts, axis=0)


def _finish_kernel(ps_ref, w_ref, b_ref, o_ref, *, ratio):
    B = ps_ref.shape[0]
    C = ps_ref.shape[2]
    pf = ps_ref[...] * (1.0 / ratio)

    diff = pf[:, :, None, :] - pf[:, None, :, :]
    d2 = jnp.sum(diff * diff, axis=-1)

    a = d2[:, :, :, None]
    bj = d2[:, :, None, :]
    jidx = jax.lax.broadcasted_iota(jnp.int32, (1, 1, _N, _N), 3)
    midx = jax.lax.broadcasted_iota(jnp.int32, (1, 1, _N, _N), 2)
    beats = (bj < a) | ((bj == a) & (jidx < midx))
    rank = jnp.sum(beats.astype(jnp.float32), axis=-1)
    adj = (rank <= 2.5).astype(jnp.float32)

    c = jnp.sum(adj, axis=1) * (1.0 / ((3.0 + 1e-6) * _N))
    g = jnp.sum(c[:, :, None] * pf, axis=1)
    mean_pf = jnp.sum(pf, axis=1) * (1.0 / _N)
    out = jax.lax.dot_general(
        g, w_ref[...], (((1,), (1,)), ((), ())),
        preferred_element_type=jnp.float32)
    o_ref[...] = out + b_ref[...] + mean_pf


@jax.jit
def kernel(x, W, b):
    B, L, C = x.shape
    BB = 4
    ps = pl.pallas_call(
        _partsum_kernel,
        grid=(B // BB,),
        in_specs=[pl.BlockSpec((BB, L, C), lambda i: (i, 0, 0))],
        out_specs=pl.BlockSpec((BB, _N, C), lambda i: (i, 0, 0)),
        out_shape=jax.ShapeDtypeStruct((B, _N, C), x.dtype),
        compiler_params=pltpu.CompilerParams(
            dimension_semantics=("arbitrary",),
        ),
    )(x)

    import functools
    out = pl.pallas_call(
        functools.partial(_finish_kernel, ratio=L // _N),
        in_specs=[
            pl.BlockSpec((B, _N, C), lambda: (0, 0, 0)),
            pl.BlockSpec((C, C), lambda: (0, 0)),
            pl.BlockSpec((1, C), lambda: (0, 0)),
        ],
        out_specs=pl.BlockSpec((B, C), lambda: (0, 0)),
        out_shape=jax.ShapeDtypeStruct((B, C), x.dtype),
    )(ps, W, b.reshape(1, C))
    return out


# speedup vs baseline: 1.0899x; 1.0779x over previous
import functools

import jax
import jax.numpy as jnp
from jax.experimental import pallas as pl
from jax.experimental.pallas import tpu as pltpu

_N = 6


def _partsum_kernel(x_ref, o_ref):
    BB, L, C = x_ref.shape
    ratio = L // _N
    for bb in range(BB):
        xb = x_ref[bb]
        parts = [
            jnp.sum(xb[i * ratio:(i + 1) * ratio, :], axis=0, keepdims=True)
            for i in range(_N)
        ]
        o_ref[bb] = jnp.concatenate(parts, axis=0)


def _finish_kernel(ps_ref, w_ref, b_ref, o_ref, *, ratio):
    B, N, C = ps_ref.shape
    pf = ps_ref[...] * (1.0 / ratio)
    p = [pf[:, n, :] for n in range(N)]

    cols = [[None] * N for _ in range(N)]
    zero = jnp.zeros((B, 1), jnp.float32)
    for n in range(N):
        cols[n][n] = zero
        for m in range(n + 1, N):
            d = p[n] - p[m]
            s = jnp.sum(d * d, axis=-1, keepdims=True)
            cols[n][m] = s
            cols[m][n] = s
    D = jnp.concatenate(
        [cols[n][m] for n in range(N) for m in range(N)], axis=1)
    Dt = D.T
    row = [Dt[i:i + 1, :] for i in range(N * N)]

    deg = []
    for m in range(N):
        dm = jnp.zeros((1, B), jnp.float32)
        for n in range(N):
            r = jnp.zeros((1, B), jnp.float32)
            for j in range(N):
                if j == m:
                    continue
                if j < m:
                    r += (row[n * N + j] <= row[n * N + m]).astype(jnp.float32)
                else:
                    r += (row[n * N + j] < row[n * N + m]).astype(jnp.float32)
            dm += (r <= 2.5).astype(jnp.float32)
        deg.append(dm)
    Cmat = jnp.concatenate(deg, axis=0)
    c = Cmat.T * (1.0 / ((3.0 + 1e-6) * N))

    g = c[:, 0:1] * p[0]
    for m in range(1, N):
        g = g + c[:, m:m + 1] * p[m]
    mean_pf = jnp.sum(pf, axis=1) * (1.0 / N)
    out = jax.lax.dot_general(
        g, w_ref[...], (((1,), (1,)), ((), ())),
        preferred_element_type=jnp.float32)
    o_ref[...] = out + b_ref[...] + mean_pf


@jax.jit
def kernel(x, W, b):
    B, L, C = x.shape
    BB = 4
    ps = pl.pallas_call(
        _partsum_kernel,
        grid=(B // BB,),
        in_specs=[pl.BlockSpec((BB, L, C), lambda i: (i, 0, 0))],
        out_specs=pl.BlockSpec((BB, _N, C), lambda i: (i, 0, 0)),
        out_shape=jax.ShapeDtypeStruct((B, _N, C), x.dtype),
        compiler_params=pltpu.CompilerParams(
            dimension_semantics=("arbitrary",),
        ),
    )(x)

    out = pl.pallas_call(
        functools.partial(_finish_kernel, ratio=L // _N),
        in_specs=[
            pl.BlockSpec((B, _N, C), lambda: (0, 0, 0)),
            pl.BlockSpec((C, C), lambda: (0, 0)),
            pl.BlockSpec((1, C), lambda: (0, 0)),
        ],
        out_specs=pl.BlockSpec((B, C), lambda: (0, 0)),
        out_shape=jax.ShapeDtypeStruct((B, C), x.dtype),
    )(ps, W, b.reshape(1, C))
    return out
